# baseline (speedup 1.0000x reference)
import functools

import numpy as np
import jax
import jax.numpy as jnp
from jax import lax
from jax.experimental import pallas as pl
from jax.experimental.pallas import tpu as pltpu
from jax.experimental.pallas import tpu_sc as plsc

N_F = 768
NPW = 256
KDIM = 4 * NPW
B = 16384
T = 32
L1 = 1024
NC, NS, L = 2, 16, 16
NW = NC * NS
ROWS_PER_W = B // NW
CHUNK = 64
VECS_PER_CHUNK = CHUNK * T // L

PAD_COL = 4 * (N_F & (NPW - 1)) + (N_F >> 8)
_FEAT_OF_COL = (np.arange(KDIM) >> 2) + ((np.arange(KDIM) & 3) << 8)


def _sc_counts_body(stm_hbm, nstm_hbm, cs_hbm, cn_hbm, idx_v, cnt_v):
    wid = lax.axis_index("s") * NC + lax.axis_index("c")
    base_row = wid * ROWS_PER_W
    lane = lax.iota(jnp.int32, L)
    zeros16 = jnp.zeros((L,), jnp.int32)

    def zero_body(i, c):
        cnt_v[pl.ds(i * L, L)] = zeros16
        return c
    lax.fori_loop(0, CHUNK * NPW // L, zero_body, 0)

    def scatter_pass(sign):
        def scat(j, c):
            e = j * L + lane
            idx16 = idx_v[pl.ds(j * L, L)]
            word = idx16 & (NPW - 1)
            plane = lax.shift_right_logical(idx16, 8)
            val = lax.shift_left(jnp.full((L,), sign, jnp.int32), plane * 8)
            off = (e >> 5) * NPW + word
            plsc.addupdate_scatter(cnt_v, [off], val)
            return c
        lax.fori_loop(0, VECS_PER_CHUNK, scat, 0)

    for src, dst in ((stm_hbm, cs_hbm), (nstm_hbm, cn_hbm)):
        def chunk_body(c, _, src=src, dst=dst):
            row0 = base_row + c * CHUNK
            pltpu.sync_copy(src.at[pl.ds(row0 * T, CHUNK * T)], idx_v)
            scatter_pass(1)
            pltpu.sync_copy(cnt_v, dst.at[pl.ds(row0 * NPW, CHUNK * NPW)])
            scatter_pass(-1)
            return 0
        lax.fori_loop(0, ROWS_PER_W // CHUNK, chunk_body, 0)


@functools.cache
def _sc_counts():
    return pl.kernel(
        _sc_counts_body,
        out_type=(
            jax.ShapeDtypeStruct((B * NPW,), jnp.int32),
            jax.ShapeDtypeStruct((B * NPW,), jnp.int32),
        ),
        mesh=plsc.VectorSubcoreMesh(core_axis_name="c", subcore_axis_name="s"),
        scratch_types=[
            pltpu.VMEM((CHUNK * T,), jnp.int32),
            pltpu.VMEM((CHUNK * NPW,), jnp.int32),
        ],
        compiler_params=pltpu.CompilerParams(needs_layout_passes=False),
    )


BB = 512


def _tc_body(cs_ref, cn_ref, tab_ref, bias_ref, w_ref, bh_ref, out_ref):
    tab = tab_ref[...]
    cs = cs_ref[...].astype(jnp.bfloat16)
    cn = cn_ref[...].astype(jnp.bfloat16)
    bias = bias_ref[...]
    emb_s = jnp.dot(cs, tab, preferred_element_type=jnp.float32)
    emb_s = jnp.clip(emb_s + bias, 0.0, 1.0)
    emb_n = jnp.dot(cn, tab, preferred_element_type=jnp.float32)
    emb_n = jnp.clip(emb_n + bias, 0.0, 1.0)
    w = w_ref[...]
    hs = lax.dot_general(emb_s, w[:, :L1], (((1,), (1,)), ((), ())),
                         preferred_element_type=jnp.float32)
    hn = lax.dot_general(emb_n, w[:, L1:], (((1,), (1,)), ((), ())),
                         preferred_element_type=jnp.float32)
    heads = hs + hn + bh_ref[...]
    n_pad = cs_ref[...][:, PAD_COL:PAD_COL + 1].astype(jnp.int32)
    n_active = T - n_pad
    bucket = jnp.clip((n_active - 2) >> 2, 0, 7)
    hsel = jnp.where(
        lax.broadcasted_iota(jnp.int32, (BB, 8), 1) == bucket, heads, 0.0)
    out_ref[...] = jnp.sum(hsel, axis=1, keepdims=True)


_tc_forward = pl.pallas_call(
    _tc_body,
    grid=(B // BB,),
    in_specs=[
        pl.BlockSpec((BB, KDIM), lambda i: (i, 0)),
        pl.BlockSpec((BB, KDIM), lambda i: (i, 0)),
        pl.BlockSpec((KDIM, L1), lambda i: (0, 0)),
        pl.BlockSpec((1, L1), lambda i: (0, 0)),
        pl.BlockSpec((8, 2 * L1), lambda i: (0, 0)),
        pl.BlockSpec((1, 8), lambda i: (0, 0)),
    ],
    out_specs=pl.BlockSpec((BB, 1), lambda i: (i, 0)),
    out_shape=jax.ShapeDtypeStruct((B, 1), jnp.float32),
)


def kernel(stm_indices, nstm_indices, table, input_bias, W_hidden, b_hidden):
    stm_flat = stm_indices.reshape(-1).astype(jnp.int32)
    nstm_flat = nstm_indices.reshape(-1).astype(jnp.int32)
    cs_flat, cn_flat = _sc_counts()(stm_flat, nstm_flat)
    cs8 = lax.bitcast_convert_type(
        cs_flat.reshape(B, NPW), jnp.int8).reshape(B, KDIM)
    cn8 = lax.bitcast_convert_type(
        cn_flat.reshape(B, NPW), jnp.int8).reshape(B, KDIM)
    tab = jnp.zeros((KDIM, L1), jnp.float32).at[:N_F].set(table[:N_F])
    tab = tab[_FEAT_OF_COL].astype(jnp.bfloat16)
    return _tc_forward(cs8, cn8, tab, input_bias.reshape(1, L1),
                       W_hidden, b_hidden.reshape(1, 8))


# speedup vs baseline: 2.6317x; 2.6317x over previous
import functools

import jax
import jax.numpy as jnp
from jax import lax
from jax.experimental import pallas as pl
from jax.experimental.pallas import tpu as pltpu
from jax.experimental.pallas import tpu_sc as plsc

N_F = 768
NPW = 256
KDIM = 4 * NPW
B = 16384
T = 32
L1 = 1024
NC, NS, L = 2, 16, 16
NW = NC * NS
ROWS_PER_W = B // NW
CHUNK = 64
VECS_PER_CHUNK = CHUNK * T // L


def _sc_counts_body(stm_hbm, nstm_hbm, cs_hbm, cn_hbm, idx_v, cnt_v):
    wid = lax.axis_index("s") * NC + lax.axis_index("c")
    base_row = wid * ROWS_PER_W
    lane = lax.iota(jnp.int32, L)
    zeros16 = jnp.zeros((L,), jnp.int32)

    def zero_body(i, c):
        cnt_v[pl.ds(i * L, L)] = zeros16
        return c
    lax.fori_loop(0, CHUNK * NPW // L, zero_body, 0)

    def scatter_pass(sign):
        def scat(j, c):
            e = j * L + lane
            idx16 = idx_v[pl.ds(j * L, L)]
            word = idx16 & (NPW - 1)
            plane = lax.shift_right_logical(idx16, 8)
            val = lax.shift_left(jnp.full((L,), sign, jnp.int32), plane * 8)
            off = (e >> 5) * NPW + word
            plsc.addupdate_scatter(cnt_v, [off], val)
            return c
        lax.fori_loop(0, VECS_PER_CHUNK, scat, 0)

    for src, dst in ((stm_hbm, cs_hbm), (nstm_hbm, cn_hbm)):
        def chunk_body(c, _, src=src, dst=dst):
            row0 = base_row + c * CHUNK
            pltpu.sync_copy(src.at[pl.ds(row0 * T, CHUNK * T)], idx_v)
            scatter_pass(1)
            pltpu.sync_copy(cnt_v, dst.at[pl.ds(row0 * NPW, CHUNK * NPW)])
            scatter_pass(-1)
            return 0
        lax.fori_loop(0, ROWS_PER_W // CHUNK, chunk_body, 0)


@functools.cache
def _sc_counts():
    return pl.kernel(
        _sc_counts_body,
        out_type=(
            jax.ShapeDtypeStruct((B * NPW,), jnp.int32),
            jax.ShapeDtypeStruct((B * NPW,), jnp.int32),
        ),
        mesh=plsc.VectorSubcoreMesh(core_axis_name="c", subcore_axis_name="s"),
        scratch_types=[
            pltpu.VMEM((CHUNK * T,), jnp.int32),
            pltpu.VMEM((CHUNK * NPW,), jnp.int32),
        ],
        compiler_params=pltpu.CompilerParams(needs_layout_passes=False),
    )


BB = 512


def _unpacked_matmul(c32, tab):
    acc = None
    for p in range(4):
        plane = lax.shift_right_logical(c32, 8 * p) if p else c32
        if p < 3:
            plane = plane & 255
        part = jnp.dot(plane.astype(jnp.bfloat16), tab[NPW * p:NPW * (p + 1)],
                       preferred_element_type=jnp.float32)
        acc = part if acc is None else acc + part
    return acc


def _tc_body(cs_ref, cn_ref, tab_ref, bias_ref, w_ref, bh_ref, out_ref):
    tab = tab_ref[...]
    cs = cs_ref[...]
    cn = cn_ref[...]
    bias = bias_ref[...]
    emb_s = jnp.clip(_unpacked_matmul(cs, tab) + bias, 0.0, 1.0)
    emb_n = jnp.clip(_unpacked_matmul(cn, tab) + bias, 0.0, 1.0)
    w = w_ref[...]
    hs = lax.dot_general(emb_s, w[:, :L1], (((1,), (1,)), ((), ())),
                         preferred_element_type=jnp.float32)
    hn = lax.dot_general(emb_n, w[:, L1:], (((1,), (1,)), ((), ())),
                         preferred_element_type=jnp.float32)
    heads = hs + hn + bh_ref[...]
    n_pad = lax.shift_right_logical(cs[:, 0:1], 24)
    n_active = T - n_pad
    bucket = jnp.clip((n_active - 2) >> 2, 0, 7)
    hsel = jnp.where(
        lax.broadcasted_iota(jnp.int32, (BB, 8), 1) == bucket, heads, 0.0)
    out_ref[...] = jnp.sum(hsel, axis=1, keepdims=True)


_tc_forward = pl.pallas_call(
    _tc_body,
    grid=(B // BB,),
    in_specs=[
        pl.BlockSpec((BB, NPW), lambda i: (i, 0)),
        pl.BlockSpec((BB, NPW), lambda i: (i, 0)),
        pl.BlockSpec((KDIM, L1), lambda i: (0, 0)),
        pl.BlockSpec((1, L1), lambda i: (0, 0)),
        pl.BlockSpec((8, 2 * L1), lambda i: (0, 0)),
        pl.BlockSpec((1, 8), lambda i: (0, 0)),
    ],
    out_specs=pl.BlockSpec((BB, 1), lambda i: (i, 0)),
    out_shape=jax.ShapeDtypeStruct((B, 1), jnp.float32),
)


def kernel(stm_indices, nstm_indices, table, input_bias, W_hidden, b_hidden):
    stm_flat = stm_indices.reshape(-1).astype(jnp.int32)
    nstm_flat = nstm_indices.reshape(-1).astype(jnp.int32)
    cs_flat, cn_flat = _sc_counts()(stm_flat, nstm_flat)
    cs = cs_flat.reshape(B, NPW)
    cn = cn_flat.reshape(B, NPW)
    tab = jnp.zeros((KDIM, L1), jnp.float32).at[:N_F].set(table[:N_F])
    tab = tab.astype(jnp.bfloat16)
    return _tc_forward(cs, cn, tab, input_bias.reshape(1, L1),
                       W_hidden, b_hidden.reshape(1, 8))


# speedup vs baseline: 2.6606x; 1.0110x over previous
import functools

import jax
import jax.numpy as jnp
from jax import lax
from jax.experimental import pallas as pl
from jax.experimental.pallas import tpu as pltpu
from jax.experimental.pallas import tpu_sc as plsc

N_F = 768
NPW = 256
KDIM = 4 * NPW
B = 16384
T = 32
L1 = 1024
NC, NS, L = 2, 16, 16
NW = NC * NS
ROWS_PER_W = B // NW
CHUNK = 64
VECS_PER_CHUNK = CHUNK * T // L


def _sc_counts_body(stm_hbm, nstm_hbm, cs_hbm, cn_hbm, idx_v, cnt_v):
    wid = lax.axis_index("s") * NC + lax.axis_index("c")
    base_row = wid * ROWS_PER_W
    lane = lax.iota(jnp.int32, L)
    zeros16 = jnp.zeros((L,), jnp.int32)

    def zero_body(i, c):
        cnt_v[pl.ds(i * L, L)] = zeros16
        return c
    lax.fori_loop(0, CHUNK * NPW // L, zero_body, 0)

    def scatter_pass(sign):
        def scat(j, c):
            e = j * L + lane
            idx16 = idx_v[pl.ds(j * L, L)]
            word = idx16 & (NPW - 1)
            plane = lax.shift_right_logical(idx16, 8)
            val = lax.shift_left(jnp.full((L,), sign, jnp.int32), plane * 8)
            off = (e >> 5) * NPW + word
            plsc.addupdate_scatter(cnt_v, [off], val)
            return c
        lax.fori_loop(0, VECS_PER_CHUNK, scat, 0)

    for src, dst in ((stm_hbm, cs_hbm), (nstm_hbm, cn_hbm)):
        def chunk_body(c, _, src=src, dst=dst):
            row0 = base_row + c * CHUNK
            pltpu.sync_copy(src.at[pl.ds(row0 * T, CHUNK * T)], idx_v)
            scatter_pass(1)
            pltpu.sync_copy(cnt_v, dst.at[pl.ds(row0 * NPW, CHUNK * NPW)])
            scatter_pass(-1)
            return 0
        lax.fori_loop(0, ROWS_PER_W // CHUNK, chunk_body, 0)


@functools.cache
def _sc_counts():
    return pl.kernel(
        _sc_counts_body,
        out_type=(
            jax.ShapeDtypeStruct((B * NPW,), jnp.int32),
            jax.ShapeDtypeStruct((B * NPW,), jnp.int32),
        ),
        mesh=plsc.VectorSubcoreMesh(core_axis_name="c", subcore_axis_name="s"),
        scratch_types=[
            pltpu.VMEM((CHUNK * T,), jnp.int32),
            pltpu.VMEM((CHUNK * NPW,), jnp.int32),
        ],
        compiler_params=pltpu.CompilerParams(needs_layout_passes=False),
    )


BB = 1024


def _unpacked_matmul(c32, tab):
    acc = None
    for p in range(4):
        plane = lax.shift_right_logical(c32, 8 * p) if p else c32
        if p < 3:
            plane = plane & 255
        part = jnp.dot(plane.astype(jnp.bfloat16), tab[NPW * p:NPW * (p + 1)],
                       preferred_element_type=jnp.float32)
        acc = part if acc is None else acc + part
    return acc


def _tc_body(cs_ref, cn_ref, tab_ref, bias_ref, w_ref, bh_ref, out_ref):
    tab = tab_ref[...]
    cs = cs_ref[...]
    cn = cn_ref[...]
    bias = bias_ref[...]
    emb_s = jnp.clip(_unpacked_matmul(cs, tab) + bias, 0.0, 1.0)
    emb_n = jnp.clip(_unpacked_matmul(cn, tab) + bias, 0.0, 1.0)
    w = w_ref[...]
    hs = lax.dot_general(emb_s, w[:, :L1], (((1,), (1,)), ((), ())),
                         preferred_element_type=jnp.float32)
    hn = lax.dot_general(emb_n, w[:, L1:], (((1,), (1,)), ((), ())),
                         preferred_element_type=jnp.float32)
    heads = hs + hn + bh_ref[...]
    n_pad = lax.shift_right_logical(cs[:, 0:1], 24)
    n_active = T - n_pad
    bucket = jnp.clip((n_active - 2) >> 2, 0, 7)
    hsel = jnp.where(
        lax.broadcasted_iota(jnp.int32, (BB, 8), 1) == bucket, heads, 0.0)
    out_ref[...] = jnp.sum(hsel, axis=1, keepdims=True)


_tc_forward = pl.pallas_call(
    _tc_body,
    grid=(B // BB,),
    in_specs=[
        pl.BlockSpec((BB, NPW), lambda i: (i, 0)),
        pl.BlockSpec((BB, NPW), lambda i: (i, 0)),
        pl.BlockSpec((KDIM, L1), lambda i: (0, 0)),
        pl.BlockSpec((1, L1), lambda i: (0, 0)),
        pl.BlockSpec((8, 2 * L1), lambda i: (0, 0)),
        pl.BlockSpec((1, 8), lambda i: (0, 0)),
    ],
    out_specs=pl.BlockSpec((BB, 1), lambda i: (i, 0)),
    out_shape=jax.ShapeDtypeStruct((B, 1), jnp.float32),
)


def kernel(stm_indices, nstm_indices, table, input_bias, W_hidden, b_hidden):
    stm_flat = stm_indices.reshape(-1).astype(jnp.int32)
    nstm_flat = nstm_indices.reshape(-1).astype(jnp.int32)
    cs_flat, cn_flat = _sc_counts()(stm_flat, nstm_flat)
    cs = cs_flat.reshape(B, NPW)
    cn = cn_flat.reshape(B, NPW)
    tab = jnp.zeros((KDIM, L1), jnp.float32).at[:N_F].set(table[:N_F])
    tab = tab.astype(jnp.bfloat16)
    return _tc_forward(cs, cn, tab, input_bias.reshape(1, L1),
                       W_hidden, b_hidden.reshape(1, 8))


# speedup vs baseline: 2.8823x; 1.0833x over previous
import functools

import jax
import jax.numpy as jnp
from jax import lax
from jax.experimental import pallas as pl
from jax.experimental.pallas import tpu as pltpu
from jax.experimental.pallas import tpu_sc as plsc

N_F = 768
NPW = 256
KDIM = 3 * NPW
B = 16384
T = 32
L1 = 1024
NC, NS, L = 2, 16, 16
NW = NC * NS
ROWS_PER_W = B // NW
CHUNK = 64
VECS_PER_CHUNK = CHUNK * T // L


def _sc_counts_body(stm_hbm, nstm_hbm, cs_hbm, cn_hbm, idx_v, cnt_v):
    wid = lax.axis_index("s") * NC + lax.axis_index("c")
    base_row = wid * ROWS_PER_W
    lane = lax.iota(jnp.int32, L)
    zeros16 = jnp.zeros((L,), jnp.int32)

    def zero_body(i, c):
        cnt_v[pl.ds(i * L, L)] = zeros16
        return c
    lax.fori_loop(0, CHUNK * NPW // L, zero_body, 0)

    def scatter_pass(sign):
        def scat(j, c):
            e = j * L + lane
            idx16 = idx_v[pl.ds(j * L, L)]
            word = idx16 & (NPW - 1)
            plane = lax.shift_right_logical(idx16, 8)
            val = lax.shift_left(jnp.full((L,), sign, jnp.int32), plane * 8)
            off = (e >> 5) * NPW + word
            plsc.addupdate_scatter(cnt_v, [off], val)
            return c
        lax.fori_loop(0, VECS_PER_CHUNK, scat, 0)

    for src, dst in ((stm_hbm, cs_hbm), (nstm_hbm, cn_hbm)):
        def chunk_body(c, _, src=src, dst=dst):
            row0 = base_row + c * CHUNK
            pltpu.sync_copy(src.at[pl.ds(row0 * T, CHUNK * T)], idx_v)
            scatter_pass(1)
            pltpu.sync_copy(cnt_v, dst.at[pl.ds(row0 * NPW, CHUNK * NPW)])
            scatter_pass(-1)
            return 0
        lax.fori_loop(0, ROWS_PER_W // CHUNK, chunk_body, 0)


@functools.cache
def _sc_counts():
    return pl.kernel(
        _sc_counts_body,
        out_type=(
            jax.ShapeDtypeStruct((B * NPW,), jnp.int32),
            jax.ShapeDtypeStruct((B * NPW,), jnp.int32),
        ),
        mesh=plsc.VectorSubcoreMesh(core_axis_name="c", subcore_axis_name="s"),
        scratch_types=[
            pltpu.VMEM((CHUNK * T,), jnp.int32),
            pltpu.VMEM((CHUNK * NPW,), jnp.int32),
        ],
        compiler_params=pltpu.CompilerParams(needs_layout_passes=False),
    )


BB = 1024


def _unpacked_matmul(c32, tab):
    acc = None
    for p in range(3):
        plane = lax.shift_right_logical(c32, 8 * p) if p else c32
        plane = plane & 255
        part = jnp.dot(plane.astype(jnp.bfloat16), tab[NPW * p:NPW * (p + 1)],
                       preferred_element_type=jnp.float32)
        acc = part if acc is None else acc + part
    return acc


def _tc_body(cs_ref, cn_ref, tab_ref, bias_ref, w_ref, bh_ref, out_ref):
    tab = tab_ref[...]
    cs = cs_ref[...]
    cn = cn_ref[...]
    bias = bias_ref[...]
    emb_s = jnp.clip(_unpacked_matmul(cs, tab) + bias, 0.0, 1.0)
    emb_n = jnp.clip(_unpacked_matmul(cn, tab) + bias, 0.0, 1.0)
    w = w_ref[...]
    hs = lax.dot_general(emb_s, w[:, :L1], (((1,), (1,)), ((), ())),
                         preferred_element_type=jnp.float32)
    hn = lax.dot_general(emb_n, w[:, L1:], (((1,), (1,)), ((), ())),
                         preferred_element_type=jnp.float32)
    heads = hs + hn + bh_ref[...]
    n_pad = lax.shift_right_logical(cs[:, 0:1], 24)
    n_active = T - n_pad
    bucket = jnp.clip((n_active - 2) >> 2, 0, 7)
    hsel = jnp.where(
        lax.broadcasted_iota(jnp.int32, (BB, 8), 1) == bucket, heads, 0.0)
    out_ref[...] = jnp.sum(hsel, axis=1, keepdims=True)


_tc_forward = pl.pallas_call(
    _tc_body,
    grid=(B // BB,),
    in_specs=[
        pl.BlockSpec((BB, NPW), lambda i: (i, 0)),
        pl.BlockSpec((BB, NPW), lambda i: (i, 0)),
        pl.BlockSpec((KDIM, L1), lambda i: (0, 0)),
        pl.BlockSpec((1, L1), lambda i: (0, 0)),
        pl.BlockSpec((8, 2 * L1), lambda i: (0, 0)),
        pl.BlockSpec((1, 8), lambda i: (0, 0)),
    ],
    out_specs=pl.BlockSpec((BB, 1), lambda i: (i, 0)),
    out_shape=jax.ShapeDtypeStruct((B, 1), jnp.float32),
)


def kernel(stm_indices, nstm_indices, table, input_bias, W_hidden, b_hidden):
    stm_flat = stm_indices.reshape(-1).astype(jnp.int32)
    nstm_flat = nstm_indices.reshape(-1).astype(jnp.int32)
    cs_flat, cn_flat = _sc_counts()(stm_flat, nstm_flat)
    cs = cs_flat.reshape(B, NPW)
    cn = cn_flat.reshape(B, NPW)
    tab = table[:N_F].astype(jnp.bfloat16)
    return _tc_forward(cs, cn, tab, input_bias.reshape(1, L1),
                       W_hidden, b_hidden.reshape(1, 8))


# speedup vs baseline: 2.8953x; 1.0045x over previous
import functools

import jax
import jax.numpy as jnp
from jax import lax
from jax.experimental import pallas as pl
from jax.experimental.pallas import tpu as pltpu
from jax.experimental.pallas import tpu_sc as plsc

N_F = 768
NPW = 256
KDIM = 3 * NPW
B = 16384
T = 32
L1 = 1024
NC, NS, L = 2, 16, 16
NW = NC * NS
ROWS_PER_W = B // NW
CHUNK = 64
VECS_PER_CHUNK = CHUNK * T // L


def _sc_counts_body(stm_hbm, nstm_hbm, cs_hbm, cn_hbm, idx_v, cnt_v):
    wid = lax.axis_index("s") * NC + lax.axis_index("c")
    base_row = wid * ROWS_PER_W
    lane = lax.iota(jnp.int32, L)
    zeros16 = jnp.zeros((L,), jnp.int32)

    def zero_body(i, c):
        cnt_v[pl.ds(i * L, L)] = zeros16
        return c
    lax.fori_loop(0, CHUNK * NPW // L, zero_body, 0)

    def scatter_pass(sign):
        def scat(j, c):
            e = j * L + lane
            idx16 = idx_v[pl.ds(j * L, L)]
            word = idx16 & (NPW - 1)
            plane = lax.shift_right_logical(idx16, 8)
            val = lax.shift_left(jnp.full((L,), sign, jnp.int32), plane * 8)
            off = (e >> 5) * NPW + word
            plsc.addupdate_scatter(cnt_v, [off], val)
            return c
        lax.fori_loop(0, VECS_PER_CHUNK, scat, 0)

    for src, dst in ((stm_hbm, cs_hbm), (nstm_hbm, cn_hbm)):
        def chunk_body(c, _, src=src, dst=dst):
            row0 = base_row + c * CHUNK
            pltpu.sync_copy(src.at[pl.ds(row0 * T, CHUNK * T)], idx_v)
            scatter_pass(1)
            pltpu.sync_copy(cnt_v, dst.at[pl.ds(row0 * NPW, CHUNK * NPW)])
            scatter_pass(-1)
            return 0
        lax.fori_loop(0, ROWS_PER_W // CHUNK, chunk_body, 0)


@functools.cache
def _sc_counts():
    return pl.kernel(
        _sc_counts_body,
        out_type=(
            jax.ShapeDtypeStruct((B * NPW,), jnp.int32),
            jax.ShapeDtypeStruct((B * NPW,), jnp.int32),
        ),
        mesh=plsc.VectorSubcoreMesh(core_axis_name="c", subcore_axis_name="s"),
        scratch_types=[
            pltpu.VMEM((CHUNK * T,), jnp.int32),
            pltpu.VMEM((CHUNK * NPW,), jnp.int32),
        ],
        compiler_params=pltpu.CompilerParams(needs_layout_passes=False),
    )


BB = 1024


def _unpacked_matmul(c32, tab):
    acc = None
    for p in range(3):
        plane = lax.shift_right_logical(c32, 8 * p) if p else c32
        plane = plane & 255
        part = jnp.dot(plane.astype(jnp.bfloat16), tab[NPW * p:NPW * (p + 1)],
                       preferred_element_type=jnp.float32)
        acc = part if acc is None else acc + part
    return acc


def _tc_body(cs_ref, cn_ref, tab_ref, bias_ref, w_ref, bh_ref, out_ref):
    tab = tab_ref[...]
    cs = cs_ref[...]
    cn = cn_ref[...]
    bias = bias_ref[...]
    emb_s = jnp.clip(_unpacked_matmul(cs, tab) + bias, 0.0, 1.0)
    emb_n = jnp.clip(_unpacked_matmul(cn, tab) + bias, 0.0, 1.0)
    w = w_ref[...]
    hs = lax.dot_general(emb_s.astype(jnp.bfloat16), w[:, :L1],
                         (((1,), (1,)), ((), ())),
                         preferred_element_type=jnp.float32)
    hn = lax.dot_general(emb_n.astype(jnp.bfloat16), w[:, L1:],
                         (((1,), (1,)), ((), ())),
                         preferred_element_type=jnp.float32)
    heads = hs + hn + bh_ref[...]
    n_pad = lax.shift_right_logical(cs[:, 0:1], 24)
    n_active = T - n_pad
    bucket = jnp.clip((n_active - 2) >> 2, 0, 7)
    hsel = jnp.where(
        lax.broadcasted_iota(jnp.int32, (BB, 8), 1) == bucket, heads, 0.0)
    out_ref[...] = jnp.sum(hsel, axis=1, keepdims=True)


_tc_forward = pl.pallas_call(
    _tc_body,
    grid=(B // BB,),
    in_specs=[
        pl.BlockSpec((BB, NPW), lambda i: (i, 0)),
        pl.BlockSpec((BB, NPW), lambda i: (i, 0)),
        pl.BlockSpec((KDIM, L1), lambda i: (0, 0)),
        pl.BlockSpec((1, L1), lambda i: (0, 0)),
        pl.BlockSpec((8, 2 * L1), lambda i: (0, 0)),
        pl.BlockSpec((1, 8), lambda i: (0, 0)),
    ],
    out_specs=pl.BlockSpec((BB, 1), lambda i: (i, 0)),
    out_shape=jax.ShapeDtypeStruct((B, 1), jnp.float32),
)


def kernel(stm_indices, nstm_indices, table, input_bias, W_hidden, b_hidden):
    stm_flat = stm_indices.reshape(-1).astype(jnp.int32)
    nstm_flat = nstm_indices.reshape(-1).astype(jnp.int32)
    cs_flat, cn_flat = _sc_counts()(stm_flat, nstm_flat)
    cs = cs_flat.reshape(B, NPW)
    cn = cn_flat.reshape(B, NPW)
    tab = table[:N_F].astype(jnp.bfloat16)
    return _tc_forward(cs, cn, tab, input_bias.reshape(1, L1),
                       W_hidden.astype(jnp.bfloat16), b_hidden.reshape(1, 8))


# speedup vs baseline: 3.6736x; 1.2688x over previous
import functools

import jax
import jax.numpy as jnp
from jax import lax
from jax.experimental import pallas as pl
from jax.experimental.pallas import tpu as pltpu
from jax.experimental.pallas import tpu_sc as plsc

N_F = 768
NPW = 256
KDIM = 3 * NPW
B = 16384
T = 32
L1 = 1024
NC, NS, L = 2, 16, 16
NW = NC * NS
ROWS_PER_W = B // NW
CHUNK = 64
VECS_PER_CHUNK = CHUNK * T // L


def _sc_counts_body(stm_hbm, nstm_hbm, cs_hbm, cn_hbm, idx_v, cnt_v):
    wid = lax.axis_index("s") * NC + lax.axis_index("c")
    base_row = wid * ROWS_PER_W
    zeros16 = jnp.zeros((L,), jnp.int32)

    def zero_body(i, c):
        def zrow(v, cc, i=i):
            cnt_v[i, pl.ds(v * L, L)] = zeros16
            return cc
        return lax.fori_loop(0, NPW // L, zrow, c)
    lax.fori_loop(0, CHUNK, zero_body, 0)

    def scatter_pass(sign):
        def scat(j, c):
            row = j >> 1
            idx16 = idx_v[row, pl.ds((j & 1) * L, L)]
            word = idx16 & (NPW - 1)
            plane = lax.shift_right_logical(idx16, 8)
            val = lax.shift_left(jnp.full((L,), sign, jnp.int32), plane * 8)
            rows = jnp.full((L,), row, jnp.int32)
            plsc.addupdate_scatter(cnt_v, [rows, word], val)
            return c
        lax.fori_loop(0, VECS_PER_CHUNK, scat, 0)

    for src, dst in ((stm_hbm, cs_hbm), (nstm_hbm, cn_hbm)):
        def chunk_body(c, _, src=src, dst=dst):
            row0 = base_row + c * CHUNK
            pltpu.sync_copy(src.at[pl.ds(row0, CHUNK)], idx_v)
            scatter_pass(1)
            pltpu.sync_copy(cnt_v, dst.at[pl.ds(row0, CHUNK)])
            scatter_pass(-1)
            return 0
        lax.fori_loop(0, ROWS_PER_W // CHUNK, chunk_body, 0)


@functools.cache
def _sc_counts():
    return pl.kernel(
        _sc_counts_body,
        out_type=(
            jax.ShapeDtypeStruct((B, NPW), jnp.int32),
            jax.ShapeDtypeStruct((B, NPW), jnp.int32),
        ),
        mesh=plsc.VectorSubcoreMesh(core_axis_name="c", subcore_axis_name="s"),
        scratch_types=[
            pltpu.VMEM((CHUNK, T), jnp.int32),
            pltpu.VMEM((CHUNK, NPW), jnp.int32),
        ],
        compiler_params=pltpu.CompilerParams(needs_layout_passes=False),
    )


BB = 1024


def _unpacked_matmul(c32, tab):
    acc = None
    for p in range(3):
        plane = lax.shift_right_logical(c32, 8 * p) if p else c32
        plane = plane & 255
        part = jnp.dot(plane.astype(jnp.bfloat16), tab[NPW * p:NPW * (p + 1)],
                       preferred_element_type=jnp.float32)
        acc = part if acc is None else acc + part
    return acc


def _tc_body(cs_ref, cn_ref, tab_ref, bias_ref, w_ref, bh_ref, out_ref):
    tab = tab_ref[...]
    cs = cs_ref[...]
    cn = cn_ref[...]
    bias = bias_ref[...]
    emb_s = jnp.clip(_unpacked_matmul(cs, tab) + bias, 0.0, 1.0)
    emb_n = jnp.clip(_unpacked_matmul(cn, tab) + bias, 0.0, 1.0)
    w = w_ref[...]
    hs = lax.dot_general(emb_s.astype(jnp.bfloat16), w[:, :L1],
                         (((1,), (1,)), ((), ())),
                         preferred_element_type=jnp.float32)
    hn = lax.dot_general(emb_n.astype(jnp.bfloat16), w[:, L1:],
                         (((1,), (1,)), ((), ())),
                         preferred_element_type=jnp.float32)
    heads = hs + hn + bh_ref[...]
    n_pad = lax.shift_right_logical(cs[:, 0:1], 24)
    n_active = T - n_pad
    bucket = jnp.clip((n_active - 2) >> 2, 0, 7)
    hsel = jnp.where(
        lax.broadcasted_iota(jnp.int32, (BB, 8), 1) == bucket, heads, 0.0)
    out_ref[...] = jnp.sum(hsel, axis=1, keepdims=True)


_tc_forward = pl.pallas_call(
    _tc_body,
    grid=(B // BB,),
    in_specs=[
        pl.BlockSpec((BB, NPW), lambda i: (i, 0)),
        pl.BlockSpec((BB, NPW), lambda i: (i, 0)),
        pl.BlockSpec((KDIM, L1), lambda i: (0, 0)),
        pl.BlockSpec((1, L1), lambda i: (0, 0)),
        pl.BlockSpec((8, 2 * L1), lambda i: (0, 0)),
        pl.BlockSpec((1, 8), lambda i: (0, 0)),
    ],
    out_specs=pl.BlockSpec((BB, 1), lambda i: (i, 0)),
    out_shape=jax.ShapeDtypeStruct((B, 1), jnp.float32),
)


def kernel(stm_indices, nstm_indices, table, input_bias, W_hidden, b_hidden):
    cs, cn = _sc_counts()(stm_indices.astype(jnp.int32),
                          nstm_indices.astype(jnp.int32))
    tab = table[:N_F].astype(jnp.bfloat16)
    return _tc_forward(cs, cn, tab, input_bias.reshape(1, L1),
                       W_hidden.astype(jnp.bfloat16), b_hidden.reshape(1, 8))


# speedup vs baseline: 4.4591x; 1.2138x over previous
import functools

import jax
import jax.numpy as jnp
from jax import lax
from jax.experimental import pallas as pl
from jax.experimental.pallas import tpu as pltpu
from jax.experimental.pallas import tpu_sc as plsc

N_F = 768
NPW = 256
KDIM = 3 * NPW
B = 16384
T = 32
L1 = 1024
NC, NS, L = 2, 16, 16
NW = NC * NS
ROWS_PER_W = B // NW
CHUNK = 64
VECS_PER_CHUNK = CHUNK * T // L


def _sc_counts_body(rows_per_w, stm_hbm, nstm_hbm, cs_hbm, cn_hbm, idx_v, cnt_v):
    wid = lax.axis_index("s") * NC + lax.axis_index("c")
    base_row = wid * rows_per_w
    zeros16 = jnp.zeros((L,), jnp.int32)

    def zero_body(i, c):
        def zrow(v, cc, i=i):
            cnt_v[i, pl.ds(v * L, L)] = zeros16
            return cc
        return lax.fori_loop(0, NPW // L, zrow, c)
    lax.fori_loop(0, CHUNK, zero_body, 0)

    def scatter_pass(sign):
        def scat(j, c):
            row = j >> 1
            idx16 = idx_v[row, pl.ds((j & 1) * L, L)]
            word = idx16 & (NPW - 1)
            plane = lax.shift_right_logical(idx16, 8)
            val = lax.shift_left(jnp.full((L,), sign, jnp.int32), plane * 8)
            rows = jnp.full((L,), row, jnp.int32)
            plsc.addupdate_scatter(cnt_v, [rows, word], val)
            return c
        lax.fori_loop(0, VECS_PER_CHUNK, scat, 0)

    for src, dst in ((stm_hbm, cs_hbm), (nstm_hbm, cn_hbm)):
        def chunk_body(c, _, src=src, dst=dst):
            row0 = base_row + c * CHUNK
            pltpu.sync_copy(src.at[pl.ds(row0, CHUNK)], idx_v)
            scatter_pass(1)
            pltpu.sync_copy(cnt_v, dst.at[pl.ds(row0, CHUNK)])
            scatter_pass(-1)
            return 0
        lax.fori_loop(0, rows_per_w // CHUNK, chunk_body, 0)


@functools.cache
def _sc_counts(nrows):
    return pl.kernel(
        functools.partial(_sc_counts_body, nrows // NW),
        out_type=(
            jax.ShapeDtypeStruct((nrows, NPW), jnp.int32),
            jax.ShapeDtypeStruct((nrows, NPW), jnp.int32),
        ),
        mesh=plsc.VectorSubcoreMesh(core_axis_name="c", subcore_axis_name="s"),
        scratch_types=[
            pltpu.VMEM((CHUNK, T), jnp.int32),
            pltpu.VMEM((CHUNK, NPW), jnp.int32),
        ],
        compiler_params=pltpu.CompilerParams(needs_layout_passes=False),
    )


BB = 1024


def _unpacked_matmul(c32, tab):
    acc = None
    for p in range(3):
        plane = lax.shift_right_logical(c32, 8 * p) if p else c32
        plane = plane & 255
        part = jnp.dot(plane.astype(jnp.bfloat16), tab[NPW * p:NPW * (p + 1)],
                       preferred_element_type=jnp.float32)
        acc = part if acc is None else acc + part
    return acc


def _tc_body(cs_ref, cn_ref, tab_ref, bias_ref, w_ref, bh_ref, out_ref):
    tab = tab_ref[...]
    cs = cs_ref[...]
    cn = cn_ref[...]
    bias = bias_ref[...]
    emb_s = jnp.clip(_unpacked_matmul(cs, tab) + bias, 0.0, 1.0)
    emb_n = jnp.clip(_unpacked_matmul(cn, tab) + bias, 0.0, 1.0)
    w = w_ref[...]
    hs = lax.dot_general(emb_s.astype(jnp.bfloat16), w[:, :L1],
                         (((1,), (1,)), ((), ())),
                         preferred_element_type=jnp.float32)
    hn = lax.dot_general(emb_n.astype(jnp.bfloat16), w[:, L1:],
                         (((1,), (1,)), ((), ())),
                         preferred_element_type=jnp.float32)
    heads = hs + hn + bh_ref[...]
    n_pad = lax.shift_right_logical(cs[:, 0:1], 24)
    n_active = T - n_pad
    bucket = jnp.clip((n_active - 2) >> 2, 0, 7)
    hsel = jnp.where(
        lax.broadcasted_iota(jnp.int32, (BB, 8), 1) == bucket, heads, 0.0)
    out_ref[...] = jnp.sum(hsel, axis=1, keepdims=True)


@functools.cache
def _tc_forward(nrows):
    return pl.pallas_call(
        _tc_body,
        grid=(nrows // BB,),
        in_specs=[
            pl.BlockSpec((BB, NPW), lambda i: (i, 0)),
            pl.BlockSpec((BB, NPW), lambda i: (i, 0)),
            pl.BlockSpec((KDIM, L1), lambda i: (0, 0)),
            pl.BlockSpec((1, L1), lambda i: (0, 0)),
            pl.BlockSpec((8, 2 * L1), lambda i: (0, 0)),
            pl.BlockSpec((1, 8), lambda i: (0, 0)),
        ],
        out_specs=pl.BlockSpec((BB, 1), lambda i: (i, 0)),
        out_shape=jax.ShapeDtypeStruct((nrows, 1), jnp.float32),
    )


NSPLIT = 2


def kernel(stm_indices, nstm_indices, table, input_bias, W_hidden, b_hidden):
    tab = table[:N_F].astype(jnp.bfloat16)
    bias = input_bias.reshape(1, L1)
    w = W_hidden.astype(jnp.bfloat16)
    bh = b_hidden.reshape(1, 8)
    h = B // NSPLIT
    counts = [_sc_counts(h)(stm_indices[k * h:(k + 1) * h],
                            nstm_indices[k * h:(k + 1) * h])
              for k in range(NSPLIT)]
    outs = [_tc_forward(h)(cs, cn, tab, bias, w, bh) for cs, cn in counts]
    return jnp.concatenate(outs, axis=0)


# speedup vs baseline: 4.6095x; 1.0337x over previous
import functools

import jax
import jax.numpy as jnp
from jax import lax
from jax.experimental import pallas as pl
from jax.experimental.pallas import tpu as pltpu
from jax.experimental.pallas import tpu_sc as plsc

N_F = 768
NPW = 256
KDIM = 3 * NPW
B = 16384
T = 32
L1 = 1024
NC, NS, L = 2, 16, 16
NW = NC * NS
ROWS_PER_W = B // NW
CHUNK = 64
VECS_PER_CHUNK = CHUNK * T // L


def _sc_counts_body(rows_per_w, stm_hbm, nstm_hbm, cs_hbm, cn_hbm, idx_v, cnt_v):
    wid = lax.axis_index("s") * NC + lax.axis_index("c")
    base_row = wid * rows_per_w
    zeros16 = jnp.zeros((L,), jnp.int32)

    def zero_body(i, c):
        def zrow(v, cc, i=i):
            cnt_v[i, pl.ds(v * L, L)] = zeros16
            return cc
        return lax.fori_loop(0, NPW // L, zrow, c)
    lax.fori_loop(0, CHUNK, zero_body, 0)

    def scatter_pass(sign):
        def scat(j, c):
            row = j >> 1
            idx16 = idx_v[row, pl.ds((j & 1) * L, L)]
            word = idx16 & (NPW - 1)
            plane = lax.shift_right_logical(idx16, 8)
            val = lax.shift_left(jnp.full((L,), sign, jnp.int32), plane * 8)
            rows = jnp.full((L,), row, jnp.int32)
            plsc.addupdate_scatter(cnt_v, [rows, word], val)
            return c
        lax.fori_loop(0, VECS_PER_CHUNK, scat, 0)

    for src, dst in ((stm_hbm, cs_hbm), (nstm_hbm, cn_hbm)):
        def chunk_body(c, _, src=src, dst=dst):
            row0 = base_row + c * CHUNK
            pltpu.sync_copy(src.at[pl.ds(row0, CHUNK)], idx_v)
            scatter_pass(1)
            pltpu.sync_copy(cnt_v, dst.at[pl.ds(row0, CHUNK)])
            scatter_pass(-1)
            return 0
        lax.fori_loop(0, rows_per_w // CHUNK, chunk_body, 0)


@functools.cache
def _sc_counts(nrows):
    return pl.kernel(
        functools.partial(_sc_counts_body, nrows // NW),
        out_type=(
            jax.ShapeDtypeStruct((nrows, NPW), jnp.int32),
            jax.ShapeDtypeStruct((nrows, NPW), jnp.int32),
        ),
        mesh=plsc.VectorSubcoreMesh(core_axis_name="c", subcore_axis_name="s"),
        scratch_types=[
            pltpu.VMEM((CHUNK, T), jnp.int32),
            pltpu.VMEM((CHUNK, NPW), jnp.int32),
        ],
        compiler_params=pltpu.CompilerParams(needs_layout_passes=False),
    )


BB = 1024


def _unpacked_matmul(c32, tab):
    acc = None
    for p in range(3):
        plane = lax.shift_right_logical(c32, 8 * p) if p else c32
        plane = plane & 255
        part = jnp.dot(plane.astype(jnp.bfloat16), tab[NPW * p:NPW * (p + 1)],
                       preferred_element_type=jnp.float32)
        acc = part if acc is None else acc + part
    return acc


def _tc_body(cs_ref, cn_ref, tab_ref, bias_ref, w_ref, bh_ref, out_ref):
    tab = tab_ref[...]
    cs = cs_ref[...]
    cn = cn_ref[...]
    bias = bias_ref[...]
    emb_s = jnp.clip(_unpacked_matmul(cs, tab) + bias, 0.0, 1.0)
    emb_n = jnp.clip(_unpacked_matmul(cn, tab) + bias, 0.0, 1.0)
    w = w_ref[...]
    hs = lax.dot_general(emb_s.astype(jnp.bfloat16), w[:, :L1],
                         (((1,), (1,)), ((), ())),
                         preferred_element_type=jnp.float32)
    hn = lax.dot_general(emb_n.astype(jnp.bfloat16), w[:, L1:],
                         (((1,), (1,)), ((), ())),
                         preferred_element_type=jnp.float32)
    heads = hs + hn + bh_ref[...]
    n_pad = lax.shift_right_logical(cs[:, 0:1], 24)
    n_active = T - n_pad
    bucket = jnp.clip((n_active - 2) >> 2, 0, 7)
    hsel = jnp.where(
        lax.broadcasted_iota(jnp.int32, (BB, 8), 1) == bucket, heads, 0.0)
    out_ref[...] = jnp.sum(hsel, axis=1, keepdims=True)


@functools.cache
def _tc_forward(nrows):
    return pl.pallas_call(
        _tc_body,
        grid=(nrows // BB,),
        in_specs=[
            pl.BlockSpec((BB, NPW), lambda i: (i, 0)),
            pl.BlockSpec((BB, NPW), lambda i: (i, 0)),
            pl.BlockSpec((KDIM, L1), lambda i: (0, 0)),
            pl.BlockSpec((1, L1), lambda i: (0, 0)),
            pl.BlockSpec((8, 2 * L1), lambda i: (0, 0)),
            pl.BlockSpec((1, 8), lambda i: (0, 0)),
        ],
        out_specs=pl.BlockSpec((BB, 1), lambda i: (i, 0)),
        out_shape=jax.ShapeDtypeStruct((nrows, 1), jnp.float32),
    )


NSPLIT = 4


def kernel(stm_indices, nstm_indices, table, input_bias, W_hidden, b_hidden):
    tab = table[:N_F].astype(jnp.bfloat16)
    bias = input_bias.reshape(1, L1)
    w = W_hidden.astype(jnp.bfloat16)
    bh = b_hidden.reshape(1, 8)
    h = B // NSPLIT
    counts = [_sc_counts(h)(stm_indices[k * h:(k + 1) * h],
                            nstm_indices[k * h:(k + 1) * h])
              for k in range(NSPLIT)]
    outs = [_tc_forward(h)(cs, cn, tab, bias, w, bh) for cs, cn in counts]
    return jnp.concatenate(outs, axis=0)
